# Initial kernel scaffold; baseline (speedup 1.0000x reference)
import jax
import jax.numpy as jnp
from jax.experimental import pallas as pl


def kernel(feat, edge_index, ln_gamma, ln_beta, W1, b1, W2, b2, Wfc1, Wfc2, bfc2):
    raise NotImplementedError("write your pallas kernel here")


# speedup vs baseline: 3.7151x; 3.7151x over previous
import functools

import jax
import jax.numpy as jnp
from jax import lax
from jax.experimental import pallas as pl
from jax.experimental.pallas import tpu as pltpu
from jax.experimental.pallas import tpu_sc as plsc

N_GRAPHS = 256
P = 22
N = N_GRAPHS * P
EPG = 176
E = N_GRAPHS * EPG
ADJ_W = P * P

NUM_WORKERS = 32
GPW = N_GRAPHS // NUM_WORKERS
EPW = GPW * EPG
SLAB = GPW * ADJ_W


@functools.lru_cache(maxsize=None)
def _make_adj_sc():
    @functools.partial(
        pl.kernel,
        mesh=plsc.VectorSubcoreMesh(core_axis_name="c", subcore_axis_name="s"),
        compiler_params=pltpu.CompilerParams(needs_layout_passes=False),
        out_type=jax.ShapeDtypeStruct((N_GRAPHS * ADJ_W,), jnp.float32),
        scratch_types=[
            pltpu.VMEM((EPW,), jnp.int32),
            pltpu.VMEM((EPW,), jnp.int32),
            pltpu.VMEM((SLAB,), jnp.float32),
        ],
    )
    def _adj_sc(edge_hbm, adj_hbm, src_v, dst_v, acc_v):
        wid = lax.axis_index("s") * 2 + lax.axis_index("c")
        ebase = wid * EPW
        pltpu.sync_copy(edge_hbm.at[pl.ds(ebase, EPW)], src_v)
        pltpu.sync_copy(edge_hbm.at[pl.ds(E + ebase, EPW)], dst_v)

        def _zero(i, _):
            acc_v[pl.ds(i * 16, 16)] = jnp.zeros((16,), jnp.float32)
            return 0

        lax.fori_loop(0, SLAB // 16, _zero, 0)

        ones = jnp.ones((16,), jnp.float32)
        for gl in range(GPW):
            nbase = (wid * GPW + gl) * P
            for c in range(EPG // 16):
                off = gl * EPG + c * 16
                s = src_v[pl.ds(off, 16)] - nbase
                d = dst_v[pl.ds(off, 16)] - nbase
                addr = gl * ADJ_W + d * P + s
                plsc.addupdate_scatter(acc_v, [addr], ones)

        pltpu.sync_copy(acc_v, adj_hbm.at[pl.ds(wid * SLAB, SLAB)])

    return _adj_sc


NEG_INF = float("-inf")


def _masked_max(x, adj):
    acc = jnp.full(x.shape, NEG_INF, dtype=x.dtype)
    for j in range(P):
        aj = adj[:, :, j:j + 1] > 0.0
        xj = x[:, j:j + 1, :]
        acc = jnp.maximum(acc, jnp.where(aj, xj, NEG_INF))
    return jnp.where(acc == NEG_INF, 0.0, acc)


def _agg_ln_body(feat_ref, adj_ref, g_ref, b_ref, out_ref):
    f = feat_ref[...]
    m = jnp.mean(f, axis=-1, keepdims=True)
    v = jnp.mean((f - m) ** 2, axis=-1, keepdims=True)
    x = (f - m) * lax.rsqrt(v + 1e-5) * g_ref[...] + b_ref[...]
    out_ref[...] = x + _masked_max(x, adj_ref[...])


def _agg_body(x_ref, adj_ref, out_ref):
    x = x_ref[...]
    out_ref[...] = x + _masked_max(x, adj_ref[...])


def _agg_ln(feat3, adj3, gamma, beta):
    Gb = 8
    D = feat3.shape[-1]
    return pl.pallas_call(
        _agg_ln_body,
        grid=(N_GRAPHS // Gb,),
        in_specs=[
            pl.BlockSpec((Gb, P, D), lambda i: (i, 0, 0)),
            pl.BlockSpec((Gb, P, P), lambda i: (i, 0, 0)),
            pl.BlockSpec((1, 1, D), lambda i: (0, 0, 0)),
            pl.BlockSpec((1, 1, D), lambda i: (0, 0, 0)),
        ],
        out_specs=pl.BlockSpec((Gb, P, D), lambda i: (i, 0, 0)),
        out_shape=jax.ShapeDtypeStruct((N_GRAPHS, P, D), jnp.float32),
    )(feat3, adj3, gamma, beta)


def _agg(x3, adj3):
    Gb = 8
    D = x3.shape[-1]
    return pl.pallas_call(
        _agg_body,
        grid=(N_GRAPHS // Gb,),
        in_specs=[
            pl.BlockSpec((Gb, P, D), lambda i: (i, 0, 0)),
            pl.BlockSpec((Gb, P, P), lambda i: (i, 0, 0)),
        ],
        out_specs=pl.BlockSpec((Gb, P, D), lambda i: (i, 0, 0)),
        out_shape=jax.ShapeDtypeStruct((N_GRAPHS, P, D), jnp.float32),
    )(x3, adj3)


def _mm_body(a_ref, w_ref, b_ref, out_ref):
    out_ref[...] = lax.dot_general(
        a_ref[...], w_ref[...],
        (((1,), (1,)), ((), ())),
        preferred_element_type=jnp.float32,
    ) + b_ref[...]


def _mm(a, w, b, mt=512):
    M, K = a.shape
    Nw = w.shape[0]
    return pl.pallas_call(
        _mm_body,
        grid=(M // mt,),
        in_specs=[
            pl.BlockSpec((mt, K), lambda i: (i, 0)),
            pl.BlockSpec((Nw, K), lambda i: (0, 0)),
            pl.BlockSpec((1, Nw), lambda i: (0, 0)),
        ],
        out_specs=pl.BlockSpec((mt, Nw), lambda i: (i, 0)),
        out_shape=jax.ShapeDtypeStruct((M, Nw), jnp.float32),
    )(a, w, b)


def _fc1_body(h_ref, w_ref, out_ref):
    z = lax.dot_general(
        h_ref[...], w_ref[...],
        (((1,), (1,)), ((), ())),
        preferred_element_type=jnp.float32,
    )
    out_ref[...] = jnp.where(z >= 0, z, 0.01 * z)


def _fc1(h, w, nt=128):
    M, K = h.shape
    Nw = w.shape[0]
    nblocks = pl.cdiv(Nw, nt)
    return pl.pallas_call(
        _fc1_body,
        grid=(nblocks,),
        in_specs=[
            pl.BlockSpec((M, K), lambda i: (0, 0)),
            pl.BlockSpec((nt, K), lambda i: (i, 0)),
        ],
        out_specs=pl.BlockSpec((M, nt), lambda i: (0, i)),
        out_shape=jax.ShapeDtypeStruct((M, Nw), jnp.float32),
    )(h, w)


def _fc2_body(z_ref, w_ref, b_ref, out_ref):
    out_ref[...] = lax.dot_general(
        z_ref[...], w_ref[...],
        (((1,), (1,)), ((), ())),
        preferred_element_type=jnp.float32,
    ) + b_ref[...]


def _fc2(z, w, b):
    M, K = z.shape
    Nw = w.shape[0]
    return pl.pallas_call(
        _fc2_body,
        in_specs=[
            pl.BlockSpec((M, K), lambda: (0, 0)),
            pl.BlockSpec((Nw, K), lambda: (0, 0)),
            pl.BlockSpec((1, Nw), lambda: (0, 0)),
        ],
        out_specs=pl.BlockSpec((M, Nw), lambda: (0, 0)),
        out_shape=jax.ShapeDtypeStruct((M, Nw), jnp.float32),
    )(z, w, b)


def kernel(feat, edge_index, ln_gamma, ln_beta, W1, b1, W2, b2, Wfc1, Wfc2, bfc2):
    D_IN = feat.shape[1]
    D_H1 = W1.shape[0]
    D_H2 = W2.shape[0]

    edge_flat = edge_index.astype(jnp.int32).reshape(-1)
    adj3 = _make_adj_sc()(edge_flat).reshape(N_GRAPHS, P, P)

    a1 = _agg_ln(feat.reshape(N_GRAPHS, P, D_IN), adj3,
                 ln_gamma.reshape(1, 1, D_IN), ln_beta.reshape(1, 1, D_IN))
    h1 = _mm(a1.reshape(N, D_IN), W1, b1.reshape(1, D_H1))
    a2 = _agg(h1.reshape(N_GRAPHS, P, D_H1), adj3)
    h2 = _mm(a2.reshape(N, D_H1), W2, b2.reshape(1, D_H2))
    z = _fc1(h2.reshape(N_GRAPHS, P * D_H2), Wfc1)
    out = _fc2(z, Wfc2, bfc2.reshape(1, -1))
    return out


# speedup vs baseline: 4.8150x; 1.2961x over previous
import functools

import jax
import jax.numpy as jnp
from jax import lax
from jax.experimental import pallas as pl
from jax.experimental.pallas import tpu as pltpu
from jax.experimental.pallas import tpu_sc as plsc

N_GRAPHS = 256
P = 22
N = N_GRAPHS * P
EPG = 176
E = N_GRAPHS * EPG
ADJ_W = P * P

NUM_WORKERS = 32
GPW = N_GRAPHS // NUM_WORKERS
EPW = GPW * EPG
SLAB = GPW * ADJ_W


@functools.lru_cache(maxsize=None)
def _make_adj_sc():
    @functools.partial(
        pl.kernel,
        mesh=plsc.VectorSubcoreMesh(core_axis_name="c", subcore_axis_name="s"),
        compiler_params=pltpu.CompilerParams(needs_layout_passes=False),
        out_type=jax.ShapeDtypeStruct((N_GRAPHS * ADJ_W,), jnp.float32),
        scratch_types=[
            pltpu.VMEM((EPW,), jnp.int32),
            pltpu.VMEM((EPW,), jnp.int32),
            pltpu.VMEM((SLAB,), jnp.float32),
        ],
    )
    def _adj_sc(edge_hbm, adj_hbm, src_v, dst_v, acc_v):
        wid = lax.axis_index("s") * 2 + lax.axis_index("c")
        ebase = wid * EPW
        pltpu.sync_copy(edge_hbm.at[pl.ds(ebase, EPW)], src_v)
        pltpu.sync_copy(edge_hbm.at[pl.ds(E + ebase, EPW)], dst_v)

        def _zero(i, _):
            acc_v[pl.ds(i * 16, 16)] = jnp.zeros((16,), jnp.float32)
            return 0

        lax.fori_loop(0, SLAB // 16, _zero, 0)

        ones = jnp.ones((16,), jnp.float32)
        for gl in range(GPW):
            nbase = (wid * GPW + gl) * P
            for c in range(EPG // 16):
                off = gl * EPG + c * 16
                s = src_v[pl.ds(off, 16)] - nbase
                d = dst_v[pl.ds(off, 16)] - nbase
                addr = gl * ADJ_W + d * P + s
                plsc.addupdate_scatter(acc_v, [addr], ones)

        pltpu.sync_copy(acc_v, adj_hbm.at[pl.ds(wid * SLAB, SLAB)])

    return _adj_sc


NEG_INF = float("-inf")


def _chunks(D, ch=512):
    offs = []
    o = 0
    while o < D:
        offs.append((o, min(ch, D - o)))
        o += ch
    return offs


def _agg_loop(x_s, adj_ref, out_ref, Gb, D):
    def _one(g, c0, cw):
        adjg = adj_ref[g]
        rows = pl.ds(g * P, P)
        cols = pl.ds(c0, cw)
        acc = jnp.full((P, cw), NEG_INF, jnp.float32)
        for j in range(P):
            xj = x_s[pl.ds(g * P + j, 1), cols]
            mj = adjg[:, j:j + 1] > 0.0
            acc = jnp.where(mj, jnp.maximum(acc, xj), acc)
        agg = jnp.where(acc == NEG_INF, 0.0, acc)
        out_ref[rows, cols] = x_s[rows, cols] + agg

    for g in range(Gb):
        for (c0, cw) in _chunks(D, 512):
            _one(g, c0, cw)


def _agg_ln_body(feat_ref, adj_ref, g_ref, b_ref, out_ref, x_s):
    Gb = adj_ref.shape[0]
    f = feat_ref[...]
    m = jnp.mean(f, axis=-1, keepdims=True)
    v = jnp.mean((f - m) ** 2, axis=-1, keepdims=True)
    x_s[...] = (f - m) * lax.rsqrt(v + 1e-5) * g_ref[...] + b_ref[...]
    _agg_loop(x_s, adj_ref, out_ref, Gb, feat_ref.shape[-1])


def _agg_body(x_ref, adj_ref, out_ref):
    _agg_loop(x_ref, adj_ref, out_ref, adj_ref.shape[0], x_ref.shape[-1])


def _agg_ln(feat, adj3, gamma, beta):
    Gb = 8
    D = feat.shape[-1]
    return pl.pallas_call(
        _agg_ln_body,
        grid=(N_GRAPHS // Gb,),
        in_specs=[
            pl.BlockSpec((Gb * P, D), lambda i: (i, 0)),
            pl.BlockSpec((Gb, P, P), lambda i: (i, 0, 0)),
            pl.BlockSpec((1, D), lambda i: (0, 0)),
            pl.BlockSpec((1, D), lambda i: (0, 0)),
        ],
        out_specs=pl.BlockSpec((Gb * P, D), lambda i: (i, 0)),
        out_shape=jax.ShapeDtypeStruct((N, D), jnp.float32),
        scratch_shapes=[pltpu.VMEM((Gb * P, D), jnp.float32)],
    )(feat, adj3, gamma, beta)


def _agg(x, adj3):
    Gb = 8
    D = x.shape[-1]
    return pl.pallas_call(
        _agg_body,
        grid=(N_GRAPHS // Gb,),
        in_specs=[
            pl.BlockSpec((Gb * P, D), lambda i: (i, 0)),
            pl.BlockSpec((Gb, P, P), lambda i: (i, 0, 0)),
        ],
        out_specs=pl.BlockSpec((Gb * P, D), lambda i: (i, 0)),
        out_shape=jax.ShapeDtypeStruct((N, D), jnp.float32),
    )(x, adj3)


def _mm_body(a_ref, w_ref, b_ref, out_ref):
    out_ref[...] = lax.dot_general(
        a_ref[...], w_ref[...],
        (((1,), (1,)), ((), ())),
        preferred_element_type=jnp.float32,
    ) + b_ref[...]


def _mm(a, w, b, mt=512):
    M, K = a.shape
    Nw = w.shape[0]
    return pl.pallas_call(
        _mm_body,
        grid=(M // mt,),
        in_specs=[
            pl.BlockSpec((mt, K), lambda i: (i, 0)),
            pl.BlockSpec((Nw, K), lambda i: (0, 0)),
            pl.BlockSpec((1, Nw), lambda i: (0, 0)),
        ],
        out_specs=pl.BlockSpec((mt, Nw), lambda i: (i, 0)),
        out_shape=jax.ShapeDtypeStruct((M, Nw), jnp.float32),
    )(a, w, b)


def _fc1_body(h_ref, w_ref, out_ref):
    z = lax.dot_general(
        h_ref[...], w_ref[...],
        (((1,), (1,)), ((), ())),
        preferred_element_type=jnp.float32,
    )
    out_ref[...] = jnp.where(z >= 0, z, 0.01 * z)


def _fc1(h, w, nt=128):
    M, K = h.shape
    Nw = w.shape[0]
    nblocks = pl.cdiv(Nw, nt)
    return pl.pallas_call(
        _fc1_body,
        grid=(nblocks,),
        in_specs=[
            pl.BlockSpec((M, K), lambda i: (0, 0)),
            pl.BlockSpec((nt, K), lambda i: (i, 0)),
        ],
        out_specs=pl.BlockSpec((M, nt), lambda i: (0, i)),
        out_shape=jax.ShapeDtypeStruct((M, Nw), jnp.float32),
    )(h, w)


def _fc2_body(z_ref, w_ref, b_ref, out_ref):
    out_ref[...] = lax.dot_general(
        z_ref[...], w_ref[...],
        (((1,), (1,)), ((), ())),
        preferred_element_type=jnp.float32,
    ) + b_ref[...]


def _fc2(z, w, b):
    M, K = z.shape
    Nw = w.shape[0]
    return pl.pallas_call(
        _fc2_body,
        in_specs=[
            pl.BlockSpec((M, K), lambda: (0, 0)),
            pl.BlockSpec((Nw, K), lambda: (0, 0)),
            pl.BlockSpec((1, Nw), lambda: (0, 0)),
        ],
        out_specs=pl.BlockSpec((M, Nw), lambda: (0, 0)),
        out_shape=jax.ShapeDtypeStruct((M, Nw), jnp.float32),
    )(z, w, b)


def kernel(feat, edge_index, ln_gamma, ln_beta, W1, b1, W2, b2, Wfc1, Wfc2, bfc2):
    D_IN = feat.shape[1]
    D_H1 = W1.shape[0]
    D_H2 = W2.shape[0]

    edge_flat = edge_index.astype(jnp.int32).reshape(-1)
    adj3 = _make_adj_sc()(edge_flat).reshape(N_GRAPHS, P, P)

    a1 = _agg_ln(feat, adj3,
                 ln_gamma.reshape(1, D_IN), ln_beta.reshape(1, D_IN))
    h1 = _mm(a1, W1, b1.reshape(1, D_H1))
    a2 = _agg(h1, adj3)
    h2 = _mm(a2, W2, b2.reshape(1, D_H2))
    z = _fc1(h2.reshape(N_GRAPHS, P * D_H2), Wfc1)
    out = _fc2(z, Wfc2, bfc2.reshape(1, -1))
    return out


# speedup vs baseline: 5.6428x; 1.1719x over previous
import functools

import jax
import jax.numpy as jnp
from jax import lax
from jax.experimental import pallas as pl
from jax.experimental.pallas import tpu as pltpu
from jax.experimental.pallas import tpu_sc as plsc

N_GRAPHS = 256
P = 22
N = N_GRAPHS * P
EPG = 176
E = N_GRAPHS * EPG
ADJ_W = P * P

NUM_WORKERS = 32
GPW = N_GRAPHS // NUM_WORKERS
EPW = GPW * EPG
SLAB = GPW * ADJ_W


@functools.lru_cache(maxsize=None)
def _make_adj_sc():
    @functools.partial(
        pl.kernel,
        mesh=plsc.VectorSubcoreMesh(core_axis_name="c", subcore_axis_name="s"),
        compiler_params=pltpu.CompilerParams(needs_layout_passes=False),
        out_type=jax.ShapeDtypeStruct((N_GRAPHS * ADJ_W,), jnp.float32),
        scratch_types=[
            pltpu.VMEM((EPW,), jnp.int32),
            pltpu.VMEM((EPW,), jnp.int32),
            pltpu.VMEM((SLAB,), jnp.float32),
        ],
    )
    def _adj_sc(edge_hbm, adj_hbm, src_v, dst_v, acc_v):
        wid = lax.axis_index("s") * 2 + lax.axis_index("c")
        ebase = wid * EPW
        pltpu.sync_copy(edge_hbm.at[pl.ds(ebase, EPW)], src_v)
        pltpu.sync_copy(edge_hbm.at[pl.ds(E + ebase, EPW)], dst_v)

        def _zero(i, _):
            acc_v[pl.ds(i * 16, 16)] = jnp.zeros((16,), jnp.float32)
            return 0

        lax.fori_loop(0, SLAB // 16, _zero, 0)

        ones = jnp.ones((16,), jnp.float32)
        for gl in range(GPW):
            nbase = (wid * GPW + gl) * P
            for c in range(EPG // 16):
                off = gl * EPG + c * 16
                s = src_v[pl.ds(off, 16)] - nbase
                d = dst_v[pl.ds(off, 16)] - nbase
                addr = gl * ADJ_W + d * P + s
                plsc.addupdate_scatter(acc_v, [addr], ones)

        pltpu.sync_copy(acc_v, adj_hbm.at[pl.ds(wid * SLAB, SLAB)])

    return _adj_sc


NEG_INF = float("-inf")


def _chunks(D, ch=512):
    offs = []
    o = 0
    while o < D:
        offs.append((o, min(ch, D - o)))
        o += ch
    return offs


def _agg_loop(x_s, adj_ref, out_ref, Gb, D):
    dt = out_ref.dtype
    ninf = jnp.array(NEG_INF, dt)

    def _one(g, c0, cw):
        pen = jnp.where(adj_ref[g] > 0.0, 0.0, NEG_INF).astype(dt)
        rows = pl.ds(g * P, P)
        cols = pl.ds(c0, cw)
        acc = jnp.full((P, cw), ninf, dt)
        for j in range(P):
            xj = x_s[pl.ds(g * P + j, 1), cols]
            acc = jnp.maximum(acc, xj + pen[:, j:j + 1])
        agg = jnp.where(acc == ninf, jnp.array(0.0, dt), acc)
        out_ref[rows, cols] = x_s[rows, cols] + agg

    for g in range(Gb):
        for (c0, cw) in _chunks(D, 1024):
            _one(g, c0, cw)


def _agg_ln_body(feat_ref, adj_ref, g_ref, b_ref, out_ref, x_s):
    Gb = adj_ref.shape[0]
    f = feat_ref[...]
    m = jnp.mean(f, axis=-1, keepdims=True)
    v = jnp.mean((f - m) ** 2, axis=-1, keepdims=True)
    x = (f - m) * lax.rsqrt(v + 1e-5) * g_ref[...] + b_ref[...]
    x_s[...] = x.astype(x_s.dtype)
    _agg_loop(x_s, adj_ref, out_ref, Gb, feat_ref.shape[-1])


def _agg_body(x_ref, adj_ref, out_ref):
    _agg_loop(x_ref, adj_ref, out_ref, adj_ref.shape[0], x_ref.shape[-1])


def _agg_ln(feat, adj3, gamma, beta):
    Gb = 8
    D = feat.shape[-1]
    return pl.pallas_call(
        _agg_ln_body,
        grid=(N_GRAPHS // Gb,),
        in_specs=[
            pl.BlockSpec((Gb * P, D), lambda i: (i, 0)),
            pl.BlockSpec((Gb, P, P), lambda i: (i, 0, 0)),
            pl.BlockSpec((1, D), lambda i: (0, 0)),
            pl.BlockSpec((1, D), lambda i: (0, 0)),
        ],
        out_specs=pl.BlockSpec((Gb * P, D), lambda i: (i, 0)),
        out_shape=jax.ShapeDtypeStruct((N, D), jnp.bfloat16),
        scratch_shapes=[pltpu.VMEM((Gb * P, D), jnp.bfloat16)],
    )(feat, adj3, gamma, beta)


def _agg(x, adj3):
    Gb = 8
    D = x.shape[-1]
    return pl.pallas_call(
        _agg_body,
        grid=(N_GRAPHS // Gb,),
        in_specs=[
            pl.BlockSpec((Gb * P, D), lambda i: (i, 0)),
            pl.BlockSpec((Gb, P, P), lambda i: (i, 0, 0)),
        ],
        out_specs=pl.BlockSpec((Gb * P, D), lambda i: (i, 0)),
        out_shape=jax.ShapeDtypeStruct((N, D), x.dtype),
    )(x, adj3)


def _mm_body(a_ref, w_ref, b_ref, out_ref):
    out_ref[...] = (lax.dot_general(
        a_ref[...], w_ref[...],
        (((1,), (1,)), ((), ())),
        preferred_element_type=jnp.float32,
    ) + b_ref[...]).astype(out_ref.dtype)


def _mm(a, w, b, mt=512, out_dtype=jnp.float32):
    M, K = a.shape
    Nw = w.shape[0]
    return pl.pallas_call(
        _mm_body,
        grid=(M // mt,),
        in_specs=[
            pl.BlockSpec((mt, K), lambda i: (i, 0)),
            pl.BlockSpec((Nw, K), lambda i: (0, 0)),
            pl.BlockSpec((1, Nw), lambda i: (0, 0)),
        ],
        out_specs=pl.BlockSpec((mt, Nw), lambda i: (i, 0)),
        out_shape=jax.ShapeDtypeStruct((M, Nw), out_dtype),
    )(a, w, b)


def _fc1_body(h_ref, w_ref, out_ref):
    z = lax.dot_general(
        h_ref[...], w_ref[...],
        (((1,), (1,)), ((), ())),
        preferred_element_type=jnp.float32,
    )
    out_ref[...] = jnp.where(z >= 0, z, 0.01 * z)


def _fc1(h, w, nt=128):
    M, K = h.shape
    Nw = w.shape[0]
    nblocks = pl.cdiv(Nw, nt)
    return pl.pallas_call(
        _fc1_body,
        grid=(nblocks,),
        in_specs=[
            pl.BlockSpec((M, K), lambda i: (0, 0)),
            pl.BlockSpec((nt, K), lambda i: (i, 0)),
        ],
        out_specs=pl.BlockSpec((M, nt), lambda i: (0, i)),
        out_shape=jax.ShapeDtypeStruct((M, Nw), jnp.float32),
    )(h, w)


def _fc2_body(z_ref, w_ref, b_ref, out_ref):
    out_ref[...] = lax.dot_general(
        z_ref[...], w_ref[...],
        (((1,), (1,)), ((), ())),
        preferred_element_type=jnp.float32,
    ) + b_ref[...]


def _fc2(z, w, b):
    M, K = z.shape
    Nw = w.shape[0]
    return pl.pallas_call(
        _fc2_body,
        in_specs=[
            pl.BlockSpec((M, K), lambda: (0, 0)),
            pl.BlockSpec((Nw, K), lambda: (0, 0)),
            pl.BlockSpec((1, Nw), lambda: (0, 0)),
        ],
        out_specs=pl.BlockSpec((M, Nw), lambda: (0, 0)),
        out_shape=jax.ShapeDtypeStruct((M, Nw), jnp.float32),
    )(z, w, b)


def kernel(feat, edge_index, ln_gamma, ln_beta, W1, b1, W2, b2, Wfc1, Wfc2, bfc2):
    D_IN = feat.shape[1]
    D_H1 = W1.shape[0]
    D_H2 = W2.shape[0]

    edge_flat = edge_index.astype(jnp.int32).reshape(-1)
    adj3 = _make_adj_sc()(edge_flat).reshape(N_GRAPHS, P, P)

    a1 = _agg_ln(feat, adj3,
                 ln_gamma.reshape(1, D_IN), ln_beta.reshape(1, D_IN))
    h1 = _mm(a1, W1.astype(jnp.bfloat16), b1.reshape(1, D_H1),
             out_dtype=jnp.bfloat16)
    a2 = _agg(h1, adj3)
    h2 = _mm(a2, W2.astype(jnp.bfloat16), b2.reshape(1, D_H2))
    z = _fc1(h2.reshape(N_GRAPHS, P * D_H2), Wfc1)
    out = _fc2(z, Wfc2, bfc2.reshape(1, -1))
    return out


# speedup vs baseline: 5.9522x; 1.0548x over previous
import functools

import jax
import jax.numpy as jnp
from jax import lax
from jax.experimental import pallas as pl
from jax.experimental.pallas import tpu as pltpu
from jax.experimental.pallas import tpu_sc as plsc

N_GRAPHS = 256
P = 22
N = N_GRAPHS * P
EPG = 176
E = N_GRAPHS * EPG
ADJ_W = P * P

NUM_WORKERS = 32
GPW = N_GRAPHS // NUM_WORKERS
EPW = GPW * EPG
SLAB = GPW * ADJ_W


@functools.lru_cache(maxsize=None)
def _make_adj_sc():
    @functools.partial(
        pl.kernel,
        mesh=plsc.VectorSubcoreMesh(core_axis_name="c", subcore_axis_name="s"),
        compiler_params=pltpu.CompilerParams(needs_layout_passes=False),
        out_type=jax.ShapeDtypeStruct((N_GRAPHS * ADJ_W,), jnp.float32),
        scratch_types=[
            pltpu.VMEM((EPW,), jnp.int32),
            pltpu.VMEM((EPW,), jnp.int32),
            pltpu.VMEM((SLAB,), jnp.float32),
        ],
    )
    def _adj_sc(edge_hbm, adj_hbm, src_v, dst_v, acc_v):
        wid = lax.axis_index("s") * 2 + lax.axis_index("c")
        ebase = wid * EPW
        pltpu.sync_copy(edge_hbm.at[pl.ds(ebase, EPW)], src_v)
        pltpu.sync_copy(edge_hbm.at[pl.ds(E + ebase, EPW)], dst_v)

        def _zero(i, _):
            acc_v[pl.ds(i * 16, 16)] = jnp.zeros((16,), jnp.float32)
            return 0

        lax.fori_loop(0, SLAB // 16, _zero, 0)

        ones = jnp.ones((16,), jnp.float32)
        for gl in range(GPW):
            nbase = (wid * GPW + gl) * P
            for c in range(EPG // 16):
                off = gl * EPG + c * 16
                s = src_v[pl.ds(off, 16)] - nbase
                d = dst_v[pl.ds(off, 16)] - nbase
                addr = gl * ADJ_W + d * P + s
                plsc.addupdate_scatter(acc_v, [addr], ones)

        pltpu.sync_copy(acc_v, adj_hbm.at[pl.ds(wid * SLAB, SLAB)])

    return _adj_sc


NEG_INF = float("-inf")


def _chunks(D, ch=512):
    offs = []
    o = 0
    while o < D:
        offs.append((o, min(ch, D - o)))
        o += ch
    return offs


def _agg_loop(xb_s, xf_s, adj_ref, out_ref, Gb, D):
    ninf = jnp.array(NEG_INF, jnp.bfloat16)

    def _one(g, c0, cw):
        pen = jnp.where(adj_ref[g] > 0.0, 0.0, NEG_INF).astype(jnp.bfloat16)
        rows = pl.ds(g * P, P)
        cols = pl.ds(c0, cw)
        acc = jnp.full((P, cw), ninf, jnp.bfloat16)
        for j in range(P):
            xj = xb_s[pl.ds(g * P + j, 1), cols]
            acc = jnp.maximum(acc, xj + pen[:, j:j + 1])
        agg = jnp.where(acc == ninf, jnp.array(0.0, jnp.bfloat16), acc)
        out_ref[rows, cols] = xf_s[rows, cols] + agg.astype(jnp.float32)

    for g in range(Gb):
        for (c0, cw) in _chunks(D, 1024):
            _one(g, c0, cw)


def _agg_ln_body(feat_ref, adj_ref, g_ref, b_ref, out_ref, xf_s, xb_s):
    Gb = adj_ref.shape[0]
    f = feat_ref[...]
    m = jnp.mean(f, axis=-1, keepdims=True)
    v = jnp.mean((f - m) ** 2, axis=-1, keepdims=True)
    x = (f - m) * lax.rsqrt(v + 1e-5) * g_ref[...] + b_ref[...]
    xf_s[...] = x
    xb_s[...] = x.astype(jnp.bfloat16)
    _agg_loop(xb_s, xf_s, adj_ref, out_ref, Gb, feat_ref.shape[-1])


def _agg_ln(feat, adj3, gamma, beta):
    Gb = 8
    D = feat.shape[-1]
    return pl.pallas_call(
        _agg_ln_body,
        grid=(N_GRAPHS // Gb,),
        in_specs=[
            pl.BlockSpec((Gb * P, D), lambda i: (i, 0)),
            pl.BlockSpec((Gb, P, P), lambda i: (i, 0, 0)),
            pl.BlockSpec((1, D), lambda i: (0, 0)),
            pl.BlockSpec((1, D), lambda i: (0, 0)),
        ],
        out_specs=pl.BlockSpec((Gb * P, D), lambda i: (i, 0)),
        out_shape=jax.ShapeDtypeStruct((N, D), jnp.float32),
        scratch_shapes=[
            pltpu.VMEM((Gb * P, D), jnp.float32),
            pltpu.VMEM((Gb * P, D), jnp.bfloat16),
        ],
    )(feat, adj3, gamma, beta)


def _gin_mid_body(a_ref, adj_ref, w1_ref, b1_ref, w2_ref, b2_ref, out_ref,
                  h1f_s, h1b_s, a2_s):
    Gb = adj_ref.shape[0]
    h = lax.dot_general(
        a_ref[...], w1_ref[...],
        (((1,), (1,)), ((), ())),
        preferred_element_type=jnp.float32,
    ) + b1_ref[...]
    h1f_s[...] = h
    h1b_s[...] = h.astype(jnp.bfloat16)
    _agg_loop(h1b_s, h1f_s, adj_ref, a2_s, Gb, h1f_s.shape[-1])
    out_ref[...] = lax.dot_general(
        a2_s[...], w2_ref[...],
        (((1,), (1,)), ((), ())),
        preferred_element_type=jnp.float32,
    ) + b2_ref[...]


def _gin_mid(a1, adj3, W1, b1, W2, b2):
    Gb = 16
    mt = Gb * P
    D1 = W1.shape[1]
    H1 = W1.shape[0]
    H2 = W2.shape[0]
    return pl.pallas_call(
        _gin_mid_body,
        grid=(N_GRAPHS // Gb,),
        in_specs=[
            pl.BlockSpec((mt, D1), lambda i: (i, 0)),
            pl.BlockSpec((Gb, P, P), lambda i: (i, 0, 0)),
            pl.BlockSpec((H1, D1), lambda i: (0, 0)),
            pl.BlockSpec((1, H1), lambda i: (0, 0)),
            pl.BlockSpec((H2, H1), lambda i: (0, 0)),
            pl.BlockSpec((1, H2), lambda i: (0, 0)),
        ],
        out_specs=pl.BlockSpec((mt, H2), lambda i: (i, 0)),
        out_shape=jax.ShapeDtypeStruct((N, H2), jnp.float32),
        scratch_shapes=[
            pltpu.VMEM((mt, H1), jnp.float32),
            pltpu.VMEM((mt, H1), jnp.bfloat16),
            pltpu.VMEM((mt, H1), jnp.float32),
        ],
    )(a1, adj3, W1, b1, W2, b2)


def _fc_body(h_ref, w1_ref, w2_ref, b_ref, out_ref, acc_s):
    i = pl.program_id(0)
    nb = pl.num_programs(0)
    nt = w1_ref.shape[0]
    nfc = D_FC_TOTAL
    z = lax.dot_general(
        h_ref[...], w1_ref[...],
        (((1,), (1,)), ((), ())),
        preferred_element_type=jnp.float32,
    )
    z = jnp.where(z >= 0, z, 0.01 * z)
    col = lax.broadcasted_iota(jnp.int32, (1, nt), 1) + i * nt
    valid = col < nfc
    z = jnp.where(valid, z, 0.0)
    w2 = jnp.where(valid, w2_ref[...], 0.0)
    contrib = lax.dot_general(
        z, w2,
        (((1,), (1,)), ((), ())),
        preferred_element_type=jnp.float32,
    )

    @pl.when(i == 0)
    def _():
        acc_s[...] = jnp.zeros_like(acc_s)

    acc_s[...] += contrib

    @pl.when(i == nb - 1)
    def _():
        out_ref[...] = acc_s[...] + b_ref[...]


D_FC_TOTAL = 3971


def _fc(h, wfc1, wfc2, b, nt=128):
    M, K = h.shape
    Nw = wfc1.shape[0]
    nblocks = pl.cdiv(Nw, nt)
    no = wfc2.shape[0]
    return pl.pallas_call(
        _fc_body,
        grid=(nblocks,),
        in_specs=[
            pl.BlockSpec((M, K), lambda i: (0, 0)),
            pl.BlockSpec((nt, K), lambda i: (i, 0)),
            pl.BlockSpec((no, nt), lambda i: (0, i)),
            pl.BlockSpec((1, no), lambda i: (0, 0)),
        ],
        out_specs=pl.BlockSpec((M, no), lambda i: (0, 0)),
        out_shape=jax.ShapeDtypeStruct((M, no), jnp.float32),
        scratch_shapes=[pltpu.VMEM((M, no), jnp.float32)],
    )(h, wfc1, wfc2, b)


def kernel(feat, edge_index, ln_gamma, ln_beta, W1, b1, W2, b2, Wfc1, Wfc2, bfc2):
    D_IN = feat.shape[1]
    D_H1 = W1.shape[0]
    D_H2 = W2.shape[0]

    edge_flat = edge_index.astype(jnp.int32).reshape(-1)
    adj3 = _make_adj_sc()(edge_flat).reshape(N_GRAPHS, P, P)

    a1 = _agg_ln(feat, adj3,
                 ln_gamma.reshape(1, D_IN), ln_beta.reshape(1, D_IN))
    h2 = _gin_mid(a1, adj3, W1, b1.reshape(1, D_H1), W2, b2.reshape(1, D_H2))
    out = _fc(h2.reshape(N_GRAPHS, P * D_H2), Wfc1, Wfc2, bfc2.reshape(1, -1))
    return out
